# baseline (speedup 1.0000x reference)
import functools

import jax
import jax.numpy as jnp
from jax.experimental import pallas as pl
from jax.experimental.pallas import tpu as pltpu

N = 1024
V = 100000
PAD = 0
SMOOTH_W = 0.1 / (V - 2)
CONF = 0.9

BR = 256
BV = 2048
NR = N // BR
NV = (V + BV - 1) // BV


def _loss_kernel(tgt_ref, pred_ref, out_ref, m_ref, s_ref, sum_ref, pt_ref, p0_ref):
    r = pl.program_id(0)
    v = pl.program_id(1)

    x = pred_ref[...]
    t = tgt_ref[...]

    @pl.when(v == 0)
    def _init():
        m_ref[...] = jnp.full_like(m_ref, -jnp.inf)
        s_ref[...] = jnp.zeros_like(s_ref)
        sum_ref[...] = jnp.zeros_like(sum_ref)
        pt_ref[...] = jnp.zeros_like(pt_ref)
        p0_ref[...] = x[:, 0:1]

    lanes = jax.lax.broadcasted_iota(jnp.int32, (BR, BV), 1)

    def update(x_for_exp, x_for_sum):
        m_old = m_ref[...]
        m_new = jnp.maximum(m_old, jnp.max(x_for_exp, axis=1, keepdims=True))
        s_ref[...] = (s_ref[...] * jnp.exp(m_old - m_new)
                      + jnp.sum(jnp.exp(x_for_exp - m_new), axis=1, keepdims=True))
        m_ref[...] = m_new
        sum_ref[...] += jnp.sum(x_for_sum, axis=1, keepdims=True)

    @pl.when(v < NV - 1)
    def _full():
        update(x, x)

    @pl.when(v == NV - 1)
    def _tail():
        valid = lanes < (V - (NV - 1) * BV)
        update(jnp.where(valid, x, -jnp.inf), jnp.where(valid, x, 0.0))

    pt_ref[...] += jnp.sum(jnp.where(lanes == t - v * BV, x, 0.0),
                           axis=1, keepdims=True)

    @pl.when(v == NV - 1)
    def _finish():
        lse = m_ref[...] + jnp.log(s_ref[...])
        sum_ls = sum_ref[...] - V * lse
        pt_ls = pt_ref[...] - lse
        p0_ls = p0_ref[...] - lse
        row_loss = -(SMOOTH_W * (sum_ls - p0_ls - pt_ls) + CONF * pt_ls)
        row_loss = jnp.where(t == PAD, 0.0, row_loss)
        partial = jnp.sum(row_loss, axis=(0, 1), keepdims=True) / N

        @pl.when(r == 0)
        def _():
            out_ref[...] = partial

        @pl.when(r > 0)
        def _():
            out_ref[...] += partial


@jax.jit
def _label_smoothing_loss(pred, target):
    tgt2 = target.astype(jnp.int32).reshape(N, 1)
    out = pl.pallas_call(
        _loss_kernel,
        grid=(NR, NV),
        in_specs=[
            pl.BlockSpec((BR, 1), lambda r, v: (r, 0)),
            pl.BlockSpec((BR, BV), lambda r, v: (r, v)),
        ],
        out_specs=pl.BlockSpec((1, 1), lambda r, v: (0, 0)),
        out_shape=jax.ShapeDtypeStruct((1, 1), jnp.float32),
        scratch_shapes=[
            pltpu.VMEM((BR, 1), jnp.float32),
            pltpu.VMEM((BR, 1), jnp.float32),
            pltpu.VMEM((BR, 1), jnp.float32),
            pltpu.VMEM((BR, 1), jnp.float32),
            pltpu.VMEM((BR, 1), jnp.float32),
        ],
    )(tgt2, pred)
    return out[0, 0]


def kernel(pred, target):
    return _label_smoothing_loss(pred, target)


# speedup vs baseline: 1.0238x; 1.0238x over previous
import functools

import jax
import jax.numpy as jnp
from jax.experimental import pallas as pl
from jax.experimental.pallas import tpu as pltpu

N = 1024
V = 100000
PAD = 0
SMOOTH_W = 0.1 / (V - 2)
CONF = 0.9

BR = 256
BV = 2048
NR = N // BR
NV = (V + BV - 1) // BV


def _loss_kernel(tgt_ref, pred_ref, out_ref, m_ref, s_ref, sum_ref, pt_ref, p0_ref):
    r = pl.program_id(0)
    v = pl.program_id(1)

    t = tgt_ref[...]

    @pl.when(v == 0)
    def _init():
        m_ref[...] = jnp.full_like(m_ref, -jnp.inf)
        s_ref[...] = jnp.zeros_like(s_ref)
        sum_ref[...] = jnp.zeros_like(sum_ref)
        pt_ref[...] = jnp.zeros_like(pt_ref)
        p0_ref[...] = pred_ref[:, 0:1]

    lanes = jax.lax.broadcasted_iota(jnp.int32, (BR, BV), 1)

    @pl.when(v < NV - 1)
    def _full():
        m_old = m_ref[...]
        m_new = jnp.maximum(m_old, jnp.max(pred_ref[...], axis=1, keepdims=True))
        s_ref[...] = (s_ref[...] * jnp.exp(m_old - m_new)
                      + jnp.sum(jnp.exp(pred_ref[...] - m_new),
                                axis=1, keepdims=True))
        m_ref[...] = m_new
        sum_ref[...] += jnp.sum(pred_ref[...], axis=1, keepdims=True)

    @pl.when(v == NV - 1)
    def _tail():
        valid = lanes < (V - (NV - 1) * BV)
        m_old = m_ref[...]
        m_new = jnp.maximum(
            m_old,
            jnp.max(jnp.where(valid, pred_ref[...], -jnp.inf),
                    axis=1, keepdims=True))
        s_ref[...] = (s_ref[...] * jnp.exp(m_old - m_new)
                      + jnp.sum(jnp.where(valid,
                                          jnp.exp(pred_ref[...] - m_new), 0.0),
                                axis=1, keepdims=True))
        m_ref[...] = m_new
        sum_ref[...] += jnp.sum(jnp.where(valid, pred_ref[...], 0.0),
                                axis=1, keepdims=True)

    pt_ref[...] += jnp.sum(jnp.where(lanes == t - v * BV, pred_ref[...], 0.0),
                           axis=1, keepdims=True)

    @pl.when(v == NV - 1)
    def _finish():
        lse = m_ref[...] + jnp.log(s_ref[...])
        sum_ls = sum_ref[...] - V * lse
        pt_ls = pt_ref[...] - lse
        p0_ls = p0_ref[...] - lse
        row_loss = -(SMOOTH_W * (sum_ls - p0_ls - pt_ls) + CONF * pt_ls)
        row_loss = jnp.where(t == PAD, 0.0, row_loss)
        partial = jnp.sum(row_loss, axis=(0, 1), keepdims=True) / N

        @pl.when(r == 0)
        def _():
            out_ref[...] = partial

        @pl.when(r > 0)
        def _():
            out_ref[...] += partial


@jax.jit
def _label_smoothing_loss(pred, target):
    tgt2 = target.astype(jnp.int32).reshape(N, 1)
    out = pl.pallas_call(
        _loss_kernel,
        grid=(NR, NV),
        in_specs=[
            pl.BlockSpec((BR, 1), lambda r, v: (r, 0)),
            pl.BlockSpec((BR, BV), lambda r, v: (r, v)),
        ],
        out_specs=pl.BlockSpec((1, 1), lambda r, v: (0, 0)),
        out_shape=jax.ShapeDtypeStruct((1, 1), jnp.float32),
        scratch_shapes=[
            pltpu.VMEM((BR, 1), jnp.float32),
            pltpu.VMEM((BR, 1), jnp.float32),
            pltpu.VMEM((BR, 1), jnp.float32),
            pltpu.VMEM((BR, 1), jnp.float32),
            pltpu.VMEM((BR, 1), jnp.float32),
        ],
    )(tgt2, pred)
    return out[0, 0]


def kernel(pred, target):
    return _label_smoothing_loss(pred, target)


# speedup vs baseline: 3.1951x; 3.1209x over previous
import functools

import jax
import jax.numpy as jnp
from jax.experimental import pallas as pl
from jax.experimental.pallas import tpu as pltpu

N = 1024
V = 100000
PAD = 0
SMOOTH_W = 0.1 / (V - 2)
CONF = 0.9

BVS = 1024
NVB = (V + BVS - 1) // BVS


def _loss_kernel(tgt_ref, pred_ref, out_ref, m_ref, s_ref, sum_ref, pt_ref, p0_ref):
    v = pl.program_id(0)
    t = tgt_ref[...]

    @pl.when(v == 0)
    def _init():
        m_ref[...] = jnp.full_like(m_ref, -jnp.inf)
        s_ref[...] = jnp.zeros_like(s_ref)
        sum_ref[...] = jnp.zeros_like(sum_ref)
        pt_ref[...] = jnp.zeros_like(pt_ref)
        p0_ref[...] = pred_ref[0:1, :]

    vocab_ids = v * BVS + jax.lax.broadcasted_iota(jnp.int32, (BVS, N), 0)

    @pl.when(v < NVB - 1)
    def _full():
        m_old = m_ref[...]
        m_new = jnp.maximum(m_old, jnp.max(pred_ref[...], axis=0, keepdims=True))
        s_ref[...] = (s_ref[...] * jnp.exp(m_old - m_new)
                      + jnp.sum(jnp.exp(pred_ref[...] - m_new),
                                axis=0, keepdims=True))
        m_ref[...] = m_new
        sum_ref[...] += jnp.sum(pred_ref[...], axis=0, keepdims=True)

    @pl.when(v == NVB - 1)
    def _tail():
        valid = vocab_ids < V
        m_old = m_ref[...]
        m_new = jnp.maximum(
            m_old,
            jnp.max(jnp.where(valid, pred_ref[...], -jnp.inf),
                    axis=0, keepdims=True))
        s_ref[...] = (s_ref[...] * jnp.exp(m_old - m_new)
                      + jnp.sum(jnp.where(valid,
                                          jnp.exp(pred_ref[...] - m_new), 0.0),
                                axis=0, keepdims=True))
        m_ref[...] = m_new
        sum_ref[...] += jnp.sum(jnp.where(valid, pred_ref[...], 0.0),
                                axis=0, keepdims=True)

    pt_ref[...] += jnp.sum(jnp.where(vocab_ids == t, pred_ref[...], 0.0),
                           axis=0, keepdims=True)

    @pl.when(v == NVB - 1)
    def _finish():
        lse = m_ref[...] + jnp.log(s_ref[...])
        sum_ls = sum_ref[...] - V * lse
        pt_ls = pt_ref[...] - lse
        p0_ls = p0_ref[...] - lse
        row_loss = -(SMOOTH_W * (sum_ls - p0_ls - pt_ls) + CONF * pt_ls)
        row_loss = jnp.where(t == PAD, 0.0, row_loss)
        out_ref[...] = jnp.sum(row_loss, axis=(0, 1), keepdims=True) / N


@jax.jit
def _label_smoothing_loss(pred, target):
    pred_t = pred.T
    tgt2 = target.astype(jnp.int32).reshape(1, N)
    out = pl.pallas_call(
        _loss_kernel,
        grid=(NVB,),
        in_specs=[
            pl.BlockSpec((1, N), lambda v: (0, 0)),
            pl.BlockSpec((BVS, N), lambda v: (v, 0)),
        ],
        out_specs=pl.BlockSpec((1, 1), lambda v: (0, 0)),
        out_shape=jax.ShapeDtypeStruct((1, 1), jnp.float32),
        scratch_shapes=[
            pltpu.VMEM((1, N), jnp.float32),
            pltpu.VMEM((1, N), jnp.float32),
            pltpu.VMEM((1, N), jnp.float32),
            pltpu.VMEM((1, N), jnp.float32),
            pltpu.VMEM((1, N), jnp.float32),
        ],
    )(tgt2, pred_t)
    return out[0, 0]


def kernel(pred, target):
    return _label_smoothing_loss(pred, target)


# speedup vs baseline: 3.2864x; 1.0286x over previous
import functools

import jax
import jax.numpy as jnp
from jax.experimental import pallas as pl
from jax.experimental.pallas import tpu as pltpu

N = 1024
V = 100000
PAD = 0
SMOOTH_W = 0.1 / (V - 2)
CONF = 0.9

BVS = 1024
NVB = (V + BVS - 1) // BVS


def _loss_kernel(tgt_ref, pred_ref, out_ref, m_ref, s_ref, sum_ref, pt_ref, p0_ref):
    v = pl.program_id(0)
    t = tgt_ref[...]

    @pl.when(v == 0)
    def _init():
        m_ref[...] = jnp.full_like(m_ref, -jnp.inf)
        s_ref[...] = jnp.zeros_like(s_ref)
        sum_ref[...] = jnp.zeros_like(sum_ref)
        pt_ref[...] = jnp.zeros_like(pt_ref)
        p0_ref[...] = pred_ref[0:1, :]

    CH = 8
    t_rel = t - v * BVS
    chunk_iota = jax.lax.broadcasted_iota(jnp.int32, (CH, N), 0)
    acc_max = jnp.full((CH, N), -jnp.inf, dtype=jnp.float32)
    acc_sum = jnp.zeros((CH, N), dtype=jnp.float32)
    acc_pt = jnp.zeros((CH, N), dtype=jnp.float32)
    for c in range(BVS // CH):
        x = pred_ref[c * CH:(c + 1) * CH, :]
        acc_max = jnp.maximum(acc_max, x)
        acc_sum = acc_sum + x
        acc_pt = acc_pt + jnp.where(chunk_iota + (c * CH) == t_rel, x, 0.0)
    blk_max = jnp.max(acc_max, axis=0, keepdims=True)
    blk_sum = jnp.sum(acc_sum, axis=0, keepdims=True)
    blk_pt = jnp.sum(acc_pt, axis=0, keepdims=True)

    m_old = m_ref[...]

    @pl.when(v < NVB - 1)
    def _full():
        m_new = jnp.maximum(m_old, blk_max)
        s_ref[...] = (s_ref[...] * jnp.exp(m_old - m_new)
                      + jnp.sum(jnp.exp(pred_ref[...] - m_new),
                                axis=0, keepdims=True))
        m_ref[...] = m_new
        sum_ref[...] += blk_sum
        pt_ref[...] += blk_pt

    @pl.when(v == NVB - 1)
    def _tail():
        vocab_ids = v * BVS + jax.lax.broadcasted_iota(jnp.int32, (BVS, N), 0)
        valid = vocab_ids < V
        m_new = jnp.maximum(
            m_old,
            jnp.max(jnp.where(valid, pred_ref[...], -jnp.inf),
                    axis=0, keepdims=True))
        s_ref[...] = (s_ref[...] * jnp.exp(m_old - m_new)
                      + jnp.sum(jnp.where(valid,
                                          jnp.exp(pred_ref[...] - m_new), 0.0),
                                axis=0, keepdims=True))
        m_ref[...] = m_new
        sum_ref[...] += jnp.sum(jnp.where(valid, pred_ref[...], 0.0),
                                axis=0, keepdims=True)
        pt_ref[...] += jnp.sum(jnp.where(vocab_ids == t, pred_ref[...], 0.0),
                               axis=0, keepdims=True)

    @pl.when(v == NVB - 1)
    def _finish():
        lse = m_ref[...] + jnp.log(s_ref[...])
        sum_ls = sum_ref[...] - V * lse
        pt_ls = pt_ref[...] - lse
        p0_ls = p0_ref[...] - lse
        row_loss = -(SMOOTH_W * (sum_ls - p0_ls - pt_ls) + CONF * pt_ls)
        row_loss = jnp.where(t == PAD, 0.0, row_loss)
        out_ref[...] = jnp.sum(row_loss, axis=(0, 1), keepdims=True) / N


@jax.jit
def _label_smoothing_loss(pred, target):
    pred_t = pred.T
    tgt2 = target.astype(jnp.int32).reshape(1, N)
    out = pl.pallas_call(
        _loss_kernel,
        grid=(NVB,),
        in_specs=[
            pl.BlockSpec((1, N), lambda v: (0, 0)),
            pl.BlockSpec((BVS, N), lambda v: (v, 0)),
        ],
        out_specs=pl.BlockSpec((1, 1), lambda v: (0, 0)),
        out_shape=jax.ShapeDtypeStruct((1, 1), jnp.float32),
        scratch_shapes=[
            pltpu.VMEM((1, N), jnp.float32),
            pltpu.VMEM((1, N), jnp.float32),
            pltpu.VMEM((1, N), jnp.float32),
            pltpu.VMEM((1, N), jnp.float32),
            pltpu.VMEM((1, N), jnp.float32),
        ],
    )(tgt2, pred_t)
    return out[0, 0]


def kernel(pred, target):
    return _label_smoothing_loss(pred, target)
